# baseline (speedup 1.0000x reference)
import functools

import jax
import jax.numpy as jnp
from jax import lax
from jax.experimental import pallas as pl
from jax.experimental.pallas import tpu as pltpu

DIM = 256
QK = 256
NWIN = 7
HEADS = 8
TOPK = 4
HH = 8
W2 = HH * HH
P2 = NWIN * NWIN
SCALE = QK ** (-0.5)
DH = QK // HEADS
DV = DIM // HEADS


def _qkv_kernel(x_ref, w_ref, b_ref, q_ref, kv_ref, vimg_ref, mean_ref):
    xb = x_ref[0].reshape(DIM, HH * NWIN * HH)
    y = jnp.dot(w_ref[...], xb, preferred_element_type=jnp.float32) + b_ref[...]
    yq = y[:QK].reshape(QK, HH, NWIN, HH)
    ykv = y[QK:].reshape(QK + DIM, HH, NWIN, HH)
    q_ref[0] = yq.transpose(2, 0, 1, 3).reshape(NWIN, QK, W2)
    kv_ref[0] = ykv.transpose(2, 0, 1, 3).reshape(NWIN, QK + DIM, W2)
    vimg_ref[0] = y[2 * QK :].reshape(DIM, HH, NWIN * HH)
    m = y[: 2 * QK].reshape(2 * QK, HH, NWIN, HH)
    m = m.sum(axis=(1, 3)) * (1.0 / W2)
    mean_ref[0, 0] = m.T


def _attn_kernel(idx_ref, q_ref, kv0_ref, kv1_ref, kv2_ref, kv3_ref, out_ref):
    del idx_ref
    q = q_ref[0, 0].reshape(HEADS, DH, W2)
    ks = []
    vs = []
    for r in (kv0_ref, kv1_ref, kv2_ref, kv3_ref):
        blk = r[0, 0]
        ks.append(blk[:QK].reshape(HEADS, DH, W2))
        vs.append(blk[QK:].reshape(HEADS, DV, W2))
    k = jnp.concatenate(ks, axis=2)
    v = jnp.concatenate(vs, axis=2)
    logits = lax.dot_general(
        q, k, (((1,), (1,)), ((0,), (0,))), preferred_element_type=jnp.float32
    ) * SCALE
    mx = logits.max(axis=2, keepdims=True)
    e = jnp.exp(logits - mx)
    a = e / e.sum(axis=2, keepdims=True)
    o = lax.dot_general(
        v, a, (((2,), (2,)), ((0,), (0,))), preferred_element_type=jnp.float32
    )
    out_ref[0, 0] = o.reshape(DIM, W2)


def _out_kernel(attn_ref, lp_ref, lb_ref, wo_ref, wb_ref, out_ref, *, h, w):
    lb = lb_ref[...].reshape(DIM, 1, 1)
    for j in range(NWIN):
        at = attn_ref[0, j * NWIN : (j + 1) * NWIN]
        at = at.reshape(NWIN, DIM, HH, HH).transpose(1, 2, 0, 3).reshape(DIM, HH, w)
        z = at + lp_ref[0][:, j * HH : (j + 1) * HH, :] + lb
        o = jnp.dot(
            wo_ref[...], z.reshape(DIM, HH * w), preferred_element_type=jnp.float32
        ) + wb_ref[...]
        out_ref[0, :, j * HH : (j + 1) * HH, :] = o.reshape(DIM, HH, w)


@jax.jit
def kernel(x, qkv_w, qkv_b, lepe_w, lepe_b, wo_w, wo_b):
    N, C, H, W = x.shape
    f32 = jnp.float32

    q, kv, vimg, means = pl.pallas_call(
        _qkv_kernel,
        grid=(N, NWIN),
        in_specs=[
            pl.BlockSpec((1, C, HH, W), lambda n, j: (n, 0, j, 0)),
            pl.BlockSpec((QK + QK + DIM, C), lambda n, j: (0, 0)),
            pl.BlockSpec((QK + QK + DIM, 1), lambda n, j: (0, 0)),
        ],
        out_specs=[
            pl.BlockSpec((1, NWIN, QK, W2), lambda n, j: (n, j, 0, 0)),
            pl.BlockSpec((1, NWIN, QK + DIM, W2), lambda n, j: (n, j, 0, 0)),
            pl.BlockSpec((1, DIM, HH, W), lambda n, j: (n, 0, j, 0)),
            pl.BlockSpec((1, 1, NWIN, 2 * QK), lambda n, j: (n, j, 0, 0)),
        ],
        out_shape=[
            jax.ShapeDtypeStruct((N, P2, QK, W2), f32),
            jax.ShapeDtypeStruct((N, P2, QK + DIM, W2), f32),
            jax.ShapeDtypeStruct((N, DIM, H, W), f32),
            jax.ShapeDtypeStruct((N, NWIN, NWIN, 2 * QK), f32),
        ],
    )(x, qkv_w, qkv_b.reshape(-1, 1))

    m = means.reshape(N, P2, 2 * QK)
    logits = jnp.einsum('npc,nqc->npq', m[..., :QK] * SCALE, m[..., QK:])
    _, idx = lax.top_k(logits, TOPK)
    idx = idx.astype(jnp.int32)

    def q_map(n, p, idx_ref):
        return (n, p, 0, 0)

    def kv_map(t):
        def im(n, p, idx_ref):
            return (n, idx_ref[n, p, t], 0, 0)

        return im

    attn_w = pl.pallas_call(
        _attn_kernel,
        grid_spec=pltpu.PrefetchScalarGridSpec(
            num_scalar_prefetch=1,
            grid=(N, P2),
            in_specs=[
                pl.BlockSpec((1, 1, QK, W2), q_map),
                pl.BlockSpec((1, 1, QK + DIM, W2), kv_map(0)),
                pl.BlockSpec((1, 1, QK + DIM, W2), kv_map(1)),
                pl.BlockSpec((1, 1, QK + DIM, W2), kv_map(2)),
                pl.BlockSpec((1, 1, QK + DIM, W2), kv_map(3)),
            ],
            out_specs=pl.BlockSpec((1, 1, DIM, W2), q_map),
        ),
        out_shape=jax.ShapeDtypeStruct((N, P2, DIM, W2), f32),
    )(idx, q, kv, kv, kv, kv)

    lepe = lax.conv_general_dilated(
        vimg, lepe_w, (1, 1), 'SAME',
        dimension_numbers=('NCHW', 'OIHW', 'NCHW'), feature_group_count=C)

    out = pl.pallas_call(
        functools.partial(_out_kernel, h=H, w=W),
        grid=(N,),
        in_specs=[
            pl.BlockSpec((1, P2, DIM, W2), lambda n: (n, 0, 0, 0)),
            pl.BlockSpec((1, DIM, H, W), lambda n: (n, 0, 0, 0)),
            pl.BlockSpec((DIM, 1), lambda n: (0, 0)),
            pl.BlockSpec((DIM, DIM), lambda n: (0, 0)),
            pl.BlockSpec((DIM, 1), lambda n: (0, 0)),
        ],
        out_specs=pl.BlockSpec((1, DIM, H, W), lambda n: (n, 0, 0, 0)),
        out_shape=jax.ShapeDtypeStruct((N, DIM, H, W), f32),
    )(
        attn_w,
        lepe,
        lepe_b.reshape(DIM, 1),
        wo_w,
        wo_b.reshape(DIM, 1),
    )
    return out
